# baseline (speedup 1.0000x reference)
import functools

import jax
import jax.numpy as jnp
from jax.experimental import pallas as pl
from jax.experimental.pallas import tpu as pltpu

ALPHA = 0.1
N = 16384
D = 64
BM = 256


def _prop_kernel(adj_ref, inp_ref, h_ref, out_ref):
    i = pl.program_id(0)
    rows = pl.ds(i * BM, BM)
    out_ref[rows, :] = (1.0 - ALPHA) * jnp.dot(
        adj_ref[...], inp_ref[...], preferred_element_type=jnp.float32
    ) + ALPHA * h_ref[rows, :]


@functools.partial(jax.jit, static_argnames=())
def kernel(input, adj, h, W):
    del W
    return pl.pallas_call(
        _prop_kernel,
        grid=(N // BM,),
        in_specs=[
            pl.BlockSpec((BM, N), lambda i: (i, 0)),
            pl.BlockSpec((N, D), lambda i: (0, 0)),
            pl.BlockSpec((N, D), lambda i: (0, 0)),
        ],
        out_specs=pl.BlockSpec((N, D), lambda i: (0, 0)),
        out_shape=jax.ShapeDtypeStruct((N, D), jnp.float32),
        compiler_params=pltpu.CompilerParams(
            dimension_semantics=("arbitrary",),
        ),
    )(adj, input, h)
